# baseline (speedup 1.0000x reference)
import jax
import jax.numpy as jnp
from jax.experimental import pallas as pl

_NUM_P = 8
_OUT_C = 64


def _fused_step(u_ref, wp_ref, bp_ref, wr_ref, br_ref, out_ref):
    u = u_ref[0]
    pooled = jnp.mean(u[:, :_OUT_C], axis=0, keepdims=True)
    logits = (
        jnp.dot(pooled, wr_ref[...], preferred_element_type=jnp.float32)
        + br_ref[...]
    )
    iota = jax.lax.broadcasted_iota(jnp.int32, (1, _NUM_P), 1)
    m1 = jnp.max(logits, axis=1, keepdims=True)
    i1 = jnp.min(jnp.where(logits == m1, iota, _NUM_P), axis=1, keepdims=True)
    masked = jnp.where(iota == i1, -jnp.inf, logits)
    m2 = jnp.max(masked, axis=1, keepdims=True)
    i2 = jnp.min(jnp.where(masked == m2, iota, _NUM_P), axis=1, keepdims=True)
    e = jnp.exp(m2 - m1)
    p1 = 1.0 / (1.0 + e)
    p2 = e / (1.0 + e)
    acc = jnp.zeros((_OUT_C, _OUT_C), jnp.float32)
    bacc = jnp.zeros((1, _OUT_C), jnp.float32)
    for p in range(_NUM_P):
        w_p = jnp.where(i1 == p, p1, 0.0) + jnp.where(i2 == p, p2, 0.0)
        acc = acc + w_p * wp_ref[p]
        bacc = bacc + w_p * bp_ref[p : p + 1, :]
    delta = jnp.dot(u, acc, preferred_element_type=jnp.float32)
    out_ref[0] = u[:, :_OUT_C] + delta + bacc


def kernel(u_t, W_prim, b_prim, W_router, b_router):
    B, N, C = u_t.shape
    br = b_router.reshape(1, _NUM_P)
    return pl.pallas_call(
        _fused_step,
        grid=(B,),
        in_specs=[
            pl.BlockSpec((1, N, C), lambda b: (b, 0, 0)),
            pl.BlockSpec((_NUM_P, C, _OUT_C), lambda b: (0, 0, 0)),
            pl.BlockSpec((_NUM_P, _OUT_C), lambda b: (0, 0)),
            pl.BlockSpec((C, _NUM_P), lambda b: (0, 0)),
            pl.BlockSpec((1, _NUM_P), lambda b: (0, 0)),
        ],
        out_specs=pl.BlockSpec((1, N, _OUT_C), lambda b: (b, 0, 0)),
        out_shape=jax.ShapeDtypeStruct((B, N, _OUT_C), jnp.float32),
    )(u_t, W_prim, b_prim, W_router, br)
